# baseline (speedup 1.0000x reference)
import functools

import jax
import jax.numpy as jnp
import numpy as np
from jax import lax
from jax.experimental import pallas as pl
from jax.experimental.pallas import tpu as pltpu
from jax.experimental.pallas import tpu_sc as plsc

R, C = 128, 32768
K = 64
NC, NS, LANES = 2, 16, 16
NW = NC * NS
ROWS_PER_W = R // NW
NV = C // LANES

_SIGN = np.uint32(0x80000000)
_ALL1 = np.uint32(0xFFFFFFFF)


def _tec_body(x_hbm, out_hbm, xbuf, cand_f, cand):
    wid = lax.axis_index("s") * NC + lax.axis_index("c")

    def do_row(k, carry):
        r = wid * ROWS_PER_W + k
        pltpu.sync_copy(x_hbm.at[r], xbuf)

        def p1(i, accs):
            a0, a1, a2, a3 = accs
            base = i * 64
            a0 = jnp.maximum(a0, xbuf[pl.ds(base, 16)])
            a1 = jnp.maximum(a1, xbuf[pl.ds(base + 16, 16)])
            a2 = jnp.maximum(a2, xbuf[pl.ds(base + 32, 16)])
            a3 = jnp.maximum(a3, xbuf[pl.ds(base + 48, 16)])
            return a0, a1, a2, a3

        ninf = jnp.full((LANES,), -jnp.inf, jnp.float32)
        a0, a1, a2, a3 = lax.fori_loop(0, C // 64, p1, (ninf, ninf, ninf, ninf))
        mv = jnp.minimum(jnp.minimum(a0, a1), jnp.minimum(a2, a3))
        lb = -jnp.max(-mv)

        def p2(i, off):
            v = xbuf[pl.ds(i * LANES, LANES)]
            m = v >= lb
            plsc.store_compressed(cand_f.at[pl.ds(off, LANES)], v, mask=m)
            pc = plsc.all_reduce_population_count(m)
            return off + pc[0]

        cnt = lax.fori_loop(0, NV, p2, jnp.int32(0))
        nv = (cnt + LANES - 1) // LANES

        lane = lax.iota(jnp.int32, LANES)

        def conv(j, c2):
            v = cand_f[pl.ds(j * LANES, LANES)]
            su = plsc.bitcast(v, jnp.uint32)
            uk = su ^ jnp.where(v >= 0.0, _SIGN, _ALL1)
            uk = jnp.where(lane < (cnt - j * LANES), uk, jnp.uint32(0))
            cand[pl.ds(j * LANES, LANES)] = uk
            return c2

        lax.fori_loop(0, nv, conv, 0)

        def bit_step(b, t):
            tc = t | lax.shift_left(np.uint32(1),
                                    np.uint32(31) - b.astype(jnp.uint32))

            def cstep(j, acc):
                u = cand[pl.ds(j * LANES, LANES)]
                return acc + (u >= tc).astype(jnp.int32)

            acc = lax.fori_loop(0, nv, cstep, jnp.zeros((LANES,), jnp.int32))
            return jnp.where(jnp.sum(acc) >= K, tc, t)

        t = lax.fori_loop(0, 32, bit_step, jnp.uint32(0))

        tv = jnp.full((LANES,), t, jnp.uint32)
        sv = jnp.where(tv < _SIGN, ~tv, tv ^ _SIGN)
        tf = plsc.bitcast(sv, jnp.float32)

        def p4(i, c2):
            v = xbuf[pl.ds(i * LANES, LANES)]
            xbuf[pl.ds(i * LANES, LANES)] = jnp.where(
                v >= tf, jnp.maximum(v, 0.0), 0.0)
            return c2

        lax.fori_loop(0, NV, p4, 0)
        pltpu.sync_copy(xbuf, out_hbm.at[r])
        return carry

    lax.fori_loop(0, ROWS_PER_W, do_row, 0)


@jax.jit
def kernel(x):
    f = pl.kernel(
        _tec_body,
        out_type=jax.ShapeDtypeStruct((R, C), jnp.float32),
        mesh=plsc.VectorSubcoreMesh(core_axis_name="c", subcore_axis_name="s"),
        compiler_params=pltpu.CompilerParams(needs_layout_passes=False),
        scratch_types=[
            pltpu.VMEM((C,), jnp.float32),
            pltpu.VMEM((C + 32,), jnp.float32),
            pltpu.VMEM((C + 32,), jnp.uint32),
        ],
    )
    return f(x)


# speedup vs baseline: 2.2841x; 2.2841x over previous
import functools

import jax
import jax.numpy as jnp
import numpy as np
from jax import lax
from jax.experimental import pallas as pl
from jax.experimental.pallas import tpu as pltpu
from jax.experimental.pallas import tpu_sc as plsc

R, C = 128, 32768
K = 64
NC, NS, LANES = 2, 16, 16
NW = NC * NS
ROWS_PER_W = R // NW
NV = C // LANES

_SIGN = np.uint32(0x80000000)
_ALL1 = np.uint32(0xFFFFFFFF)


def _tec_body(x_hbm, out_hbm, xbuf, cand_f, cand):
    wid = lax.axis_index("s") * NC + lax.axis_index("c")

    def do_row(k, carry):
        r = wid * ROWS_PER_W + k
        pltpu.sync_copy(x_hbm.at[r], xbuf)

        def p1(i, accs):
            a0, a1, a2, a3 = accs
            base = i * 64
            a0 = jnp.maximum(a0, xbuf[pl.ds(base, 16)])
            a1 = jnp.maximum(a1, xbuf[pl.ds(base + 16, 16)])
            a2 = jnp.maximum(a2, xbuf[pl.ds(base + 32, 16)])
            a3 = jnp.maximum(a3, xbuf[pl.ds(base + 48, 16)])
            return a0, a1, a2, a3

        ninf = jnp.full((LANES,), -jnp.inf, jnp.float32)
        a0, a1, a2, a3 = lax.fori_loop(0, C // 64, p1, (ninf, ninf, ninf, ninf))
        mv = jnp.minimum(jnp.minimum(a0, a1), jnp.minimum(a2, a3))
        lb = -jnp.max(-mv)

        G = 8

        def p2(i, off):
            base = i * (LANES * G)
            vs = [xbuf[pl.ds(base + g * LANES, LANES)] for g in range(G)]
            ms = [v >= lb for v in vs]
            pcs = [plsc.all_reduce_population_count(m)[0] for m in ms]
            o = off
            for g in range(G):
                plsc.store_compressed(cand_f.at[pl.ds(o, LANES)], vs[g],
                                      mask=ms[g])
                o = o + pcs[g]
            return o

        cnt = lax.fori_loop(0, NV // G, p2, jnp.int32(0))
        nv = (cnt + LANES - 1) // LANES

        lane = lax.iota(jnp.int32, LANES)

        def conv(j, c2):
            v = cand_f[pl.ds(j * LANES, LANES)]
            su = plsc.bitcast(v, jnp.uint32)
            uk = su ^ jnp.where(v >= 0.0, _SIGN, _ALL1)
            uk = jnp.where(lane < (cnt - j * LANES), uk, jnp.uint32(0))
            cand[pl.ds(j * LANES, LANES)] = uk
            return c2

        lax.fori_loop(0, nv, conv, 0)

        def bit_step(b, t):
            tc = t | lax.shift_left(np.uint32(1),
                                    np.uint32(31) - b.astype(jnp.uint32))

            def cstep(j, acc):
                u = cand[pl.ds(j * LANES, LANES)]
                return acc + (u >= tc).astype(jnp.int32)

            acc = lax.fori_loop(0, nv, cstep, jnp.zeros((LANES,), jnp.int32))
            return jnp.where(jnp.sum(acc) >= K, tc, t)

        t = lax.fori_loop(0, 32, bit_step, jnp.uint32(0))

        tv = jnp.full((LANES,), t, jnp.uint32)
        sv = jnp.where(tv < _SIGN, ~tv, tv ^ _SIGN)
        tf = plsc.bitcast(sv, jnp.float32)

        def p4(i, c2):
            base = i * (LANES * G)
            vs = [xbuf[pl.ds(base + g * LANES, LANES)] for g in range(G)]
            os_ = [jnp.where(v >= tf, jnp.maximum(v, 0.0), 0.0) for v in vs]
            for g in range(G):
                xbuf[pl.ds(base + g * LANES, LANES)] = os_[g]
            return c2

        lax.fori_loop(0, NV // G, p4, 0)
        pltpu.sync_copy(xbuf, out_hbm.at[r])
        return carry

    lax.fori_loop(0, ROWS_PER_W, do_row, 0)


@jax.jit
def kernel(x):
    f = pl.kernel(
        _tec_body,
        out_type=jax.ShapeDtypeStruct((R, C), jnp.float32),
        mesh=plsc.VectorSubcoreMesh(core_axis_name="c", subcore_axis_name="s"),
        compiler_params=pltpu.CompilerParams(needs_layout_passes=False),
        scratch_types=[
            pltpu.VMEM((C,), jnp.float32),
            pltpu.VMEM((C + 32,), jnp.float32),
            pltpu.VMEM((C + 32,), jnp.uint32),
        ],
    )
    return f(x)
